# baseline (speedup 1.0000x reference)
import functools

import jax
import jax.numpy as jnp
from jax.experimental import pallas as pl
from jax.experimental.pallas import tpu as pltpu

N = 10000
E = 320000
D_IN = 128
D_EDGE = 4
H = 8
HE = 9
L = 8
G = 32


def _node_encode_body(x_ref, w_ref, b_ref, o_ref):
    o_ref[...] = jnp.maximum(
        jnp.dot(x_ref[...], w_ref[...], preferred_element_type=jnp.float32)
        + b_ref[...], 0.0)


def _node_encode(x, W, b):
    blk = 2000
    return pl.pallas_call(
        _node_encode_body,
        grid=(N // blk,),
        in_specs=[
            pl.BlockSpec((blk, D_IN), lambda i: (i, 0)),
            pl.BlockSpec((D_IN, H), lambda i: (0, 0)),
            pl.BlockSpec((1, H), lambda i: (0, 0)),
        ],
        out_specs=pl.BlockSpec((blk, H), lambda i: (i, 0)),
        out_shape=jax.ShapeDtypeStruct((N, H), jnp.float32),
    )(x, W, b.reshape(1, H))


def _edge_pre_body(ea_ref, we_ref, be_ref, wm_ref, bm_ref, o_ref):
    e = jnp.maximum(
        jnp.dot(ea_ref[...], we_ref[...], preferred_element_type=jnp.float32)
        + be_ref[...], 0.0)
    o_ref[...] = jnp.dot(e, wm_ref[...], preferred_element_type=jnp.float32) + bm_ref[...]


def _edge_pre(edge_attr, We, be, Wm2_all, bm_all):
    blk = 4000
    return pl.pallas_call(
        _edge_pre_body,
        grid=(E // blk,),
        in_specs=[
            pl.BlockSpec((blk, D_EDGE), lambda i: (i, 0)),
            pl.BlockSpec((D_EDGE, HE), lambda i: (0, 0)),
            pl.BlockSpec((1, HE), lambda i: (0, 0)),
            pl.BlockSpec((HE, L * H), lambda i: (0, 0)),
            pl.BlockSpec((1, L * H), lambda i: (0, 0)),
        ],
        out_specs=pl.BlockSpec((blk, L * H), lambda i: (i, 0)),
        out_shape=jax.ShapeDtypeStruct((E, L * H), jnp.float32),
    )(edge_attr, We, be.reshape(1, HE), Wm2_all, bm_all)


def _msg_body(g_ref, c_ref, o_ref):
    o_ref[...] = jnp.maximum(g_ref[...] + c_ref[...], 0.0)


def _msg(g, c):
    blk = 8000
    return pl.pallas_call(
        _msg_body,
        grid=(E // blk,),
        in_specs=[
            pl.BlockSpec((blk, H), lambda i: (i, 0)),
            pl.BlockSpec((blk, H), lambda i: (i, 0)),
        ],
        out_specs=pl.BlockSpec((blk, H), lambda i: (i, 0)),
        out_shape=jax.ShapeDtypeStruct((E, H), jnp.float32),
    )(g, c)


def _update_body(h_ref, agg_ref, wu1_ref, wu2_ref, bu_ref, wm1_ref, o_ref, ow_ref):
    hn = jnp.maximum(
        jnp.dot(h_ref[...], wu1_ref[...], preferred_element_type=jnp.float32)
        + jnp.dot(agg_ref[...], wu2_ref[...], preferred_element_type=jnp.float32)
        + bu_ref[...], 0.0)
    o_ref[...] = hn
    ow_ref[...] = jnp.dot(hn, wm1_ref[...], preferred_element_type=jnp.float32)


def _update(h, agg, Wu1, Wu2, bu, Wm1_next):
    blk = 2000
    return pl.pallas_call(
        _update_body,
        grid=(N // blk,),
        in_specs=[
            pl.BlockSpec((blk, H), lambda i: (i, 0)),
            pl.BlockSpec((blk, H), lambda i: (i, 0)),
            pl.BlockSpec((H, H), lambda i: (0, 0)),
            pl.BlockSpec((H, H), lambda i: (0, 0)),
            pl.BlockSpec((1, H), lambda i: (0, 0)),
            pl.BlockSpec((H, H), lambda i: (0, 0)),
        ],
        out_specs=[
            pl.BlockSpec((blk, H), lambda i: (i, 0)),
            pl.BlockSpec((blk, H), lambda i: (i, 0)),
        ],
        out_shape=[
            jax.ShapeDtypeStruct((N, H), jnp.float32),
            jax.ShapeDtypeStruct((N, H), jnp.float32),
        ],
    )(h, agg, Wu1, Wu2, bu.reshape(1, H), Wm1_next)


def _pool_head_body(h_ref, onehot_ref, wl_ref, bl_ref, o_ref):
    oh = onehot_ref[...]
    acc = jnp.dot(oh, h_ref[...], preferred_element_type=jnp.float32)
    cnt = jnp.sum(oh, axis=1, keepdims=True)
    ge = acc / jnp.maximum(cnt, 1.0)
    o_ref[...] = jnp.dot(ge, wl_ref[...], preferred_element_type=jnp.float32) + bl_ref[...]


def _pool_head(h, onehot, W_lin, b_lin):
    return pl.pallas_call(
        _pool_head_body,
        out_shape=jax.ShapeDtypeStruct((G, 1), jnp.float32),
    )(h, onehot, W_lin, b_lin.reshape(1, 1))


def kernel(x, edge_index, edge_attr, batch,
           W_node_enc, b_node_enc, W_edge_enc, b_edge_enc,
           W_msg, b_msg, W_upd, b_upd, W_lin, b_lin):
    src = edge_index[0].astype(jnp.int32)
    dst = edge_index[1].astype(jnp.int32)
    batch = batch.astype(jnp.int32)

    h = _node_encode(x, W_node_enc, b_node_enc)
    Wm2_all = jnp.transpose(W_msg[:, H:, :], (1, 0, 2)).reshape(HE, L * H)
    bm_all = b_msg.reshape(1, L * H)
    C = _edge_pre(edge_attr, W_edge_enc, b_edge_enc, Wm2_all, bm_all)

    hW = jnp.dot(h, W_msg[0, :H, :])
    for l in range(L):
        g = jnp.take(hW, src, axis=0)
        m = _msg(g, C[:, l * H:(l + 1) * H])
        agg = jax.ops.segment_sum(m, dst, num_segments=N)
        Wm1_next = W_msg[(l + 1) % L, :H, :]
        h, hW = _update(h, agg, W_upd[l, :H, :], W_upd[l, H:, :], b_upd[l], Wm1_next)

    onehot = (batch[None, :] == jnp.arange(G, dtype=jnp.int32)[:, None]).astype(jnp.float32)
    return _pool_head(h, onehot, W_lin, b_lin)


# speedup vs baseline: 8.2627x; 8.2627x over previous
import functools

import jax
import jax.numpy as jnp
from jax import lax
from jax.experimental import pallas as pl
from jax.experimental.pallas import tpu as pltpu
from jax.experimental.pallas import tpu_sc as plsc

N = 10000
E = 320000
D_IN = 128
D_EDGE = 4
H = 8
HE = 9
L = 8
G = 32
W16 = 16

NC = 2
NS = 16
NW = NC * NS
EPW = E // NW
CH = 2000
NPAD = 10240
NPS = NPAD // NS


def _pad16(w):
    a, b = w.shape[-2], w.shape[-1]
    return jnp.pad(w, [(0, 0)] * (w.ndim - 2) + [(0, W16 - a), (0, W16 - b)])


def _node_encode_body(x_ref, w_ref, b_ref, wm_ref, h_ref, hw_ref):
    h = jnp.maximum(
        jnp.dot(x_ref[...], w_ref[...], preferred_element_type=jnp.float32)
        + b_ref[...], 0.0)
    h_ref[...] = h
    hw_ref[...] = jnp.dot(h, wm_ref[...], preferred_element_type=jnp.float32)


def _node_encode(x, Wp, bp, Wm1p0):
    blk = 2000
    return pl.pallas_call(
        _node_encode_body,
        grid=(N // blk,),
        in_specs=[
            pl.BlockSpec((blk, D_IN), lambda i: (i, 0)),
            pl.BlockSpec((D_IN, W16), lambda i: (0, 0)),
            pl.BlockSpec((1, W16), lambda i: (0, 0)),
            pl.BlockSpec((W16, W16), lambda i: (0, 0)),
        ],
        out_specs=[
            pl.BlockSpec((blk, W16), lambda i: (i, 0)),
            pl.BlockSpec((blk, W16), lambda i: (i, 0)),
        ],
        out_shape=[
            jax.ShapeDtypeStruct((N, W16), jnp.float32),
            jax.ShapeDtypeStruct((N, W16), jnp.float32),
        ],
    )(x, Wp, bp, Wm1p0)


def _edge_pre_body(ea_ref, we_ref, be_ref, wm_ref, bm_ref, *o_refs):
    e = jnp.maximum(
        jnp.dot(ea_ref[...], we_ref[...], preferred_element_type=jnp.float32)
        + be_ref[...], 0.0)
    for l in range(L):
        o_refs[l][...] = (
            jnp.dot(e, wm_ref[l], preferred_element_type=jnp.float32)
            + bm_ref[...][:, l * W16:(l + 1) * W16])


def _edge_pre(edge_attr, Wep, bep, Wm2p, bmp):
    blk = 4000
    return pl.pallas_call(
        _edge_pre_body,
        grid=(E // blk,),
        in_specs=[
            pl.BlockSpec((blk, D_EDGE), lambda i: (i, 0)),
            pl.BlockSpec((D_EDGE, HE), lambda i: (0, 0)),
            pl.BlockSpec((1, HE), lambda i: (0, 0)),
            pl.BlockSpec((L, HE, W16), lambda i: (0, 0, 0)),
            pl.BlockSpec((1, L * W16), lambda i: (0, 0)),
        ],
        out_specs=[pl.BlockSpec((blk, W16), lambda i: (i, 0))] * L,
        out_shape=[jax.ShapeDtypeStruct((E, W16), jnp.float32)] * L,
    )(edge_attr, Wep, bep, Wm2p, bmp)


def _update_body(h_ref, p_ref, wu1_ref, wu2_ref, bu_ref, wm1_ref, o_ref, ow_ref):
    agg = p_ref[0] + p_ref[1]
    hn = jnp.maximum(
        jnp.dot(h_ref[...], wu1_ref[...], preferred_element_type=jnp.float32)
        + jnp.dot(agg, wu2_ref[...], preferred_element_type=jnp.float32)
        + bu_ref[...], 0.0)
    o_ref[...] = hn
    ow_ref[...] = jnp.dot(hn, wm1_ref[...], preferred_element_type=jnp.float32)


def _update(h, partials, Wu1p, Wu2p, bup, Wm1p_next):
    blk = 2000
    return pl.pallas_call(
        _update_body,
        grid=(N // blk,),
        in_specs=[
            pl.BlockSpec((blk, W16), lambda i: (i, 0)),
            pl.BlockSpec((NC, blk, W16), lambda i: (0, i, 0)),
            pl.BlockSpec((W16, W16), lambda i: (0, 0)),
            pl.BlockSpec((W16, W16), lambda i: (0, 0)),
            pl.BlockSpec((1, W16), lambda i: (0, 0)),
            pl.BlockSpec((W16, W16), lambda i: (0, 0)),
        ],
        out_specs=[
            pl.BlockSpec((blk, W16), lambda i: (i, 0)),
            pl.BlockSpec((blk, W16), lambda i: (i, 0)),
        ],
        out_shape=[
            jax.ShapeDtypeStruct((N, W16), jnp.float32),
            jax.ShapeDtypeStruct((N, W16), jnp.float32),
        ],
    )(h, partials, Wu1p, Wu2p, bup, Wm1p_next)


def _pool_head_body(h_ref, onehot_ref, wl_ref, bl_ref, o_ref):
    oh = onehot_ref[...]
    acc = jnp.dot(oh, h_ref[...], preferred_element_type=jnp.float32)
    cnt = jnp.sum(oh, axis=1, keepdims=True)
    ge = acc / jnp.maximum(cnt, 1.0)
    o_ref[...] = jnp.dot(ge, wl_ref[...], preferred_element_type=jnp.float32) + bl_ref[...]


def _pool_head(h, onehot, Wlp, b_lin):
    return pl.pallas_call(
        _pool_head_body,
        out_shape=jax.ShapeDtypeStruct((G, 1), jnp.float32),
    )(h, onehot, Wlp, b_lin.reshape(1, 1))


def _sc_layer_body(hw_hbm, src_hbm, dst_hbm, c_hbm, zeros_hbm, out_hbm,
                   srci_v, dsti_v, g_v, c_v, acc, tbl, sem):
    c = lax.axis_index("c")
    s = lax.axis_index("s")
    wid = s * NC + c

    if True:
        pltpu.sync_copy(zeros_hbm.at[pl.ds(s * NPS, NPS)],
                        acc.at[pl.ds(s * NPS, NPS)])

        @pl.when(s < 10)
        def _stage():
            pltpu.sync_copy(hw_hbm.at[pl.ds(s * 1000, 1000)],
                            tbl.at[pl.ds(s * 1000, 1000)])

        plsc.subcore_barrier()

        for k in range(EPW // CH):
            base = wid * EPW + k * CH
            pltpu.sync_copy(src_hbm.at[pl.ds(base, CH)], srci_v)
            pltpu.sync_copy(dst_hbm.at[pl.ds(base, CH)], dsti_v)
            pltpu.sync_copy(c_hbm.at[pl.ds(base, CH)], c_v)
            pltpu.async_copy(tbl.at[srci_v], g_v, sem).wait()

            def body(r, _):
                g_v[r, :] = jnp.maximum(g_v[r, :] + c_v[r, :], 0.0)
                return 0
            lax.fori_loop(0, CH, body, 0)

            pltpu.sync_copy(g_v, acc.at[dsti_v], add=True)

        plsc.subcore_barrier()
        pltpu.sync_copy(acc.at[pl.ds(s * NPS, NPS)],
                        out_hbm.at[c, pl.ds(s * NPS, NPS)])


_sc_layer = pl.kernel(
    _sc_layer_body,
    out_type=jax.ShapeDtypeStruct((NC, NPAD, W16), jnp.float32),
    mesh=plsc.VectorSubcoreMesh(core_axis_name="c", subcore_axis_name="s"),
    compiler_params=pltpu.CompilerParams(use_tc_tiling_on_sc=False),
    scratch_types=[
        pltpu.VMEM((CH,), jnp.int32),
        pltpu.VMEM((CH,), jnp.int32),
        pltpu.VMEM((CH, W16), jnp.float32),
        pltpu.VMEM((CH, W16), jnp.float32),
        pltpu.VMEM_SHARED((NPAD, W16), jnp.float32),
        pltpu.VMEM_SHARED((N, W16), jnp.float32),
        pltpu.SemaphoreType.DMA,
    ],
)


def kernel(x, edge_index, edge_attr, batch,
           W_node_enc, b_node_enc, W_edge_enc, b_edge_enc,
           W_msg, b_msg, W_upd, b_upd, W_lin, b_lin):
    src = edge_index[0].astype(jnp.int32)
    dst = edge_index[1].astype(jnp.int32)
    batch = batch.astype(jnp.int32)

    Wnp = jnp.pad(W_node_enc, ((0, 0), (0, W16 - H)))
    bnp = jnp.pad(b_node_enc, (0, W16 - H)).reshape(1, W16)
    Wm1p = _pad16(W_msg[:, :H, :])
    Wm2p = jnp.pad(W_msg[:, H:, :], ((0, 0), (0, 0), (0, W16 - H)))
    bmp = jnp.pad(b_msg, ((0, 0), (0, W16 - H))).reshape(1, L * W16)
    Wu1p = _pad16(W_upd[:, :H, :])
    Wu2p = _pad16(W_upd[:, H:, :])
    bup = jnp.pad(b_upd, ((0, 0), (0, W16 - H)))
    Wlp = jnp.pad(W_lin, ((0, W16 - H), (0, 0)))

    h, hW = _node_encode(x, Wnp, bnp, Wm1p[0])
    C = _edge_pre(edge_attr, W_edge_enc, b_edge_enc.reshape(1, HE), Wm2p, bmp)
    zeros = jnp.zeros((NPAD, W16), jnp.float32)

    for l in range(L):
        partials = _sc_layer(hW, src, dst, C[l], zeros)
        h, hW = _update(h, partials, Wu1p[l], Wu2p[l],
                        bup[l].reshape(1, W16), Wm1p[(l + 1) % L])

    onehot = (batch[None, :] == jnp.arange(G, dtype=jnp.int32)[:, None]).astype(jnp.float32)
    return _pool_head(h, onehot, Wlp, b_lin)


# speedup vs baseline: 16.5301x; 2.0006x over previous
import functools

import jax
import jax.numpy as jnp
from jax import lax
from jax.experimental import pallas as pl
from jax.experimental.pallas import tpu as pltpu
from jax.experimental.pallas import tpu_sc as plsc

N = 10000
E = 320000
D_IN = 128
D_EDGE = 4
H = 8
HE = 9
L = 8
G = 32
W16 = 16

NC = 2
NS = 16
NW = NC * NS
EPW = E // NW
CH = 1000
NPAD = 10240
NPS = NPAD // NS


def _pad16(w):
    a, b = w.shape[-2], w.shape[-1]
    return jnp.pad(w, [(0, 0)] * (w.ndim - 2) + [(0, W16 - a), (0, W16 - b)])


def _node_encode_body(x_ref, w_ref, b_ref, wm_ref, h_ref, hw_ref):
    h = jnp.maximum(
        jnp.dot(x_ref[...], w_ref[...], preferred_element_type=jnp.float32)
        + b_ref[...], 0.0)
    h_ref[...] = h
    hw_ref[...] = jnp.dot(h, wm_ref[...], preferred_element_type=jnp.float32)


def _node_encode(x, Wp, bp, Wm1p0):
    blk = 2000
    return pl.pallas_call(
        _node_encode_body,
        grid=(N // blk,),
        in_specs=[
            pl.BlockSpec((blk, D_IN), lambda i: (i, 0)),
            pl.BlockSpec((D_IN, W16), lambda i: (0, 0)),
            pl.BlockSpec((1, W16), lambda i: (0, 0)),
            pl.BlockSpec((W16, W16), lambda i: (0, 0)),
        ],
        out_specs=[
            pl.BlockSpec((blk, W16), lambda i: (i, 0)),
            pl.BlockSpec((blk, W16), lambda i: (i, 0)),
        ],
        out_shape=[
            jax.ShapeDtypeStruct((N, W16), jnp.float32),
            jax.ShapeDtypeStruct((N, W16), jnp.float32),
        ],
    )(x, Wp, bp, Wm1p0)


def _edge_pre_body(ea_ref, we_ref, be_ref, wm_ref, bm_ref, *o_refs):
    e = jnp.maximum(
        jnp.dot(ea_ref[...], we_ref[...], preferred_element_type=jnp.float32)
        + be_ref[...], 0.0)
    for l in range(L):
        o_refs[l][...] = (
            jnp.dot(e, wm_ref[l], preferred_element_type=jnp.float32)
            + bm_ref[l])


def _edge_pre(edge_attr, Wep, bep, Wm2p, bmp):
    blk = 6400
    return pl.pallas_call(
        _edge_pre_body,
        grid=(E // blk,),
        in_specs=[
            pl.BlockSpec((blk // 8, 8 * D_EDGE), lambda i: (i, 0)),
            pl.BlockSpec((8 * D_EDGE, 8 * HE), lambda i: (0, 0)),
            pl.BlockSpec((1, 8 * HE), lambda i: (0, 0)),
            pl.BlockSpec((L, 8 * HE, 128), lambda i: (0, 0, 0)),
            pl.BlockSpec((L, 1, 128), lambda i: (0, 0, 0)),
        ],
        out_specs=[pl.BlockSpec((blk // 8, 128), lambda i: (i, 0))] * L,
        out_shape=[jax.ShapeDtypeStruct((E // 8, 128), jnp.float32)] * L,
    )(edge_attr.reshape(E // 8, 8 * D_EDGE), Wep, bep, Wm2p, bmp)


def _update_body(h_ref, p_ref, wu1_ref, wu2_ref, bu_ref, wm1_ref, o_ref, ow_ref):
    agg = p_ref[0] + p_ref[1]
    hn = jnp.maximum(
        jnp.dot(h_ref[...], wu1_ref[...], preferred_element_type=jnp.float32)
        + jnp.dot(agg, wu2_ref[...], preferred_element_type=jnp.float32)
        + bu_ref[...], 0.0)
    o_ref[...] = hn
    ow_ref[...] = jnp.dot(hn, wm1_ref[...], preferred_element_type=jnp.float32)


def _update(h, partials, Wu1p, Wu2p, bup, Wm1p_next):
    return pl.pallas_call(
        _update_body,
        grid=(1,),
        in_specs=[
            pl.BlockSpec((N, W16), lambda i: (0, 0)),
            pl.BlockSpec((NC, N, W16), lambda i: (0, 0, 0)),
            pl.BlockSpec((W16, W16), lambda i: (0, 0)),
            pl.BlockSpec((W16, W16), lambda i: (0, 0)),
            pl.BlockSpec((1, W16), lambda i: (0, 0)),
            pl.BlockSpec((W16, W16), lambda i: (0, 0)),
        ],
        out_specs=[
            pl.BlockSpec((N, W16), lambda i: (0, 0)),
            pl.BlockSpec((N, W16), lambda i: (0, 0)),
        ],
        out_shape=[
            jax.ShapeDtypeStruct((N, W16), jnp.float32),
            jax.ShapeDtypeStruct((N, W16), jnp.float32),
        ],
    )(h, partials, Wu1p, Wu2p, bup, Wm1p_next)


def _pool_head_body(h_ref, onehot_ref, wl_ref, bl_ref, o_ref):
    oh = onehot_ref[...]
    acc = jnp.dot(oh, h_ref[...], preferred_element_type=jnp.float32)
    cnt = jnp.sum(oh, axis=1, keepdims=True)
    ge = acc / jnp.maximum(cnt, 1.0)
    o_ref[...] = jnp.dot(ge, wl_ref[...], preferred_element_type=jnp.float32) + bl_ref[...]


def _pool_head(h, onehot, Wlp, b_lin):
    return pl.pallas_call(
        _pool_head_body,
        out_shape=jax.ShapeDtypeStruct((G, 1), jnp.float32),
    )(h, onehot, Wlp, b_lin.reshape(1, 1))


def _sc_layer_body(hw_hbm, src_hbm, dst_hbm, c_hbm, zeros_hbm, out_hbm,
                   srci_v0, dsti_v0, g_v0, c_v0,
                   srci_v1, dsti_v1, g_v1, c_v1,
                   acc, tbl, sem, lsem0, lsem1):
    c = lax.axis_index("c")
    s = lax.axis_index("s")
    wid = s * NC + c

    if True:
        pltpu.sync_copy(zeros_hbm.at[pl.ds(s * NPS, NPS)],
                        acc.at[pl.ds(s * NPS, NPS)])

        @pl.when(s < 10)
        def _stage():
            pltpu.sync_copy(hw_hbm.at[pl.ds(s * 1000, 1000)],
                            tbl.at[pl.ds(s * 1000, 1000)])

        plsc.subcore_barrier()

        nchunk = EPW // CH
        srci = (srci_v0, srci_v1)
        dsti = (dsti_v0, dsti_v1)
        gb = (g_v0, g_v1)
        cb = (c_v0, c_v1)
        lsem = (lsem0, lsem1)

        def prefetch(k, nb):
            base = wid * EPW + k * CH
            pltpu.async_copy(src_hbm.at[pl.ds(base, CH)], srci[nb], lsem[nb])
            pltpu.async_copy(dst_hbm.at[pl.ds(base, CH)], dsti[nb], lsem[nb])
            pltpu.async_copy(c_hbm.at[pl.ds(base // 8, CH // 8)], cb[nb], lsem[nb])

        def drain(nb):
            pltpu.make_async_copy(src_hbm.at[pl.ds(0, CH)], srci[nb], lsem[nb]).wait()
            pltpu.make_async_copy(dst_hbm.at[pl.ds(0, CH)], dsti[nb], lsem[nb]).wait()
            pltpu.make_async_copy(c_hbm.at[pl.ds(0, CH // 8)], cb[nb], lsem[nb]).wait()

        prefetch(0, 0)
        for k in range(nchunk):
            nb = k % 2
            drain(nb)
            if k + 1 < nchunk:
                prefetch(k + 1, (k + 1) % 2)
            g_v = gb[nb]
            c_v = cb[nb]
            pltpu.async_copy(tbl.at[srci[nb]], g_v, sem).wait()

            def body(i, _):
                for j in range(8):
                    r = i * 8 + j
                    g_v[r, :] = jnp.maximum(
                        g_v[r, :] + c_v[i, pl.ds(16 * j, 16)], 0.0)
                return 0
            lax.fori_loop(0, CH // 8, body, 0)

            pltpu.sync_copy(g_v, acc.at[dsti[nb]], add=True)

        plsc.subcore_barrier()
        pltpu.sync_copy(acc.at[pl.ds(s * NPS, NPS)],
                        out_hbm.at[c, pl.ds(s * NPS, NPS)])


@functools.cache
def _sc_layer_fn():
    return pl.kernel(
    _sc_layer_body,
    out_type=jax.ShapeDtypeStruct((NC, NPAD, W16), jnp.float32),
    mesh=plsc.VectorSubcoreMesh(core_axis_name="c", subcore_axis_name="s"),
    compiler_params=pltpu.CompilerParams(use_tc_tiling_on_sc=False),
    scratch_types=[
        pltpu.VMEM((CH,), jnp.int32),
        pltpu.VMEM((CH,), jnp.int32),
        pltpu.VMEM((CH, W16), jnp.float32),
        pltpu.VMEM((CH // 8, 128), jnp.float32),
        pltpu.VMEM((CH,), jnp.int32),
        pltpu.VMEM((CH,), jnp.int32),
        pltpu.VMEM((CH, W16), jnp.float32),
        pltpu.VMEM((CH // 8, 128), jnp.float32),
        pltpu.VMEM_SHARED((NPAD, W16), jnp.float32),
        pltpu.VMEM_SHARED((N, W16), jnp.float32),
        pltpu.SemaphoreType.DMA,
        pltpu.SemaphoreType.DMA,
        pltpu.SemaphoreType.DMA,
    ],
    )


def kernel(x, edge_index, edge_attr, batch,
           W_node_enc, b_node_enc, W_edge_enc, b_edge_enc,
           W_msg, b_msg, W_upd, b_upd, W_lin, b_lin):
    src = edge_index[0].astype(jnp.int32)
    dst = edge_index[1].astype(jnp.int32)
    batch = batch.astype(jnp.int32)

    Wnp = jnp.pad(W_node_enc, ((0, 0), (0, W16 - H)))
    bnp = jnp.pad(b_node_enc, (0, W16 - H)).reshape(1, W16)
    Wm1p = _pad16(W_msg[:, :H, :])
    Wm2p = jnp.pad(W_msg[:, H:, :], ((0, 0), (0, 0), (0, W16 - H)))
    bmp = jnp.pad(b_msg, ((0, 0), (0, W16 - H))).reshape(1, L * W16)
    Wu1p = _pad16(W_upd[:, :H, :])
    Wu2p = _pad16(W_upd[:, H:, :])
    bup = jnp.pad(b_upd, ((0, 0), (0, W16 - H)))
    Wlp = jnp.pad(W_lin, ((0, W16 - H), (0, 0)))

    h, hW = _node_encode(x, Wnp, bnp, Wm1p[0])
    We_bd = jnp.zeros((8 * D_EDGE, 8 * HE), jnp.float32)
    Wbig = jnp.zeros((L, 8 * HE, 128), jnp.float32)
    for j in range(8):
        We_bd = We_bd.at[D_EDGE * j:D_EDGE * (j + 1), HE * j:HE * (j + 1)].set(W_edge_enc)
        Wbig = Wbig.at[:, HE * j:HE * (j + 1), W16 * j:W16 * (j + 1)].set(Wm2p)
    be_t = jnp.tile(b_edge_enc, 8).reshape(1, 8 * HE)
    bm_t = jnp.tile(jnp.pad(b_msg, ((0, 0), (0, W16 - H))), (1, 8)).reshape(L, 1, 128)
    C = _edge_pre(edge_attr, We_bd, be_t, Wbig, bm_t)
    zeros = jnp.zeros((NPAD, W16), jnp.float32)

    for l in range(L):
        partials = _sc_layer_fn()(hW, src, dst, C[l], zeros)
        h, hW = _update(h, partials, Wu1p[l], Wu2p[l],
                        bup[l].reshape(1, W16), Wm1p[(l + 1) % L])

    onehot = (batch[None, :] == jnp.arange(G, dtype=jnp.int32)[:, None]).astype(jnp.float32)
    return _pool_head(h, onehot, Wlp, b_lin)
